# Initial kernel scaffold; baseline (speedup 1.0000x reference)
import jax
import jax.numpy as jnp
from jax.experimental import pallas as pl


def kernel(x, adj, W, b):
    raise NotImplementedError("write your pallas kernel here")


# speedup vs baseline: 11.3633x; 11.3633x over previous
import jax
import jax.numpy as jnp
from jax.experimental import pallas as pl

_N = 8192
_NFEAT = 256
_NHID = 64

_BM = 512
_LI = 256
_LJ = 2048


def _support_kernel(x_ref, w_ref, out_ref):
    out_ref[...] = jnp.dot(x_ref[...], w_ref[...],
                           preferred_element_type=jnp.float32)


def _gcz_kernel(adj_ref, sup_ref, b_ref, out_ref):
    out_ref[...] = jnp.dot(adj_ref[...], sup_ref[...],
                           preferred_element_type=jnp.float32) + b_ref[...]


def _loss_kernel(adj_ref, zi_ref, zj_ref, acc_ref):
    i = pl.program_id(0)
    j = pl.program_id(1)

    @pl.when((i == 0) & (j == 0))
    def _init():
        acc_ref[...] = jnp.zeros_like(acc_ref)

    zz = jax.lax.dot_general(
        zi_ref[...], zj_ref[...],
        dimension_numbers=(((1,), (1,)), ((), ())),
        preferred_element_type=jnp.float32)
    d = jax.nn.sigmoid(zz) - adj_ref[...]
    acc_ref[...] = acc_ref[...] + jnp.sum(d * d) * (1.0 / (_N * _N))


def kernel(x, adj, W, b):
    b2 = b.reshape(1, _NHID)

    support = pl.pallas_call(
        _support_kernel,
        out_shape=jax.ShapeDtypeStruct((_N, _NHID), jnp.float32),
    )(x, W)

    gc_z = pl.pallas_call(
        _gcz_kernel,
        grid=(_N // _BM,),
        in_specs=[
            pl.BlockSpec((_BM, _N), lambda i: (i, 0)),
            pl.BlockSpec((_N, _NHID), lambda i: (0, 0)),
            pl.BlockSpec((1, _NHID), lambda i: (0, 0)),
        ],
        out_specs=pl.BlockSpec((_BM, _NHID), lambda i: (i, 0)),
        out_shape=jax.ShapeDtypeStruct((_N, _NHID), jnp.float32),
    )(adj, support, b2)

    loss = pl.pallas_call(
        _loss_kernel,
        grid=(_N // _LI, _N // _LJ),
        in_specs=[
            pl.BlockSpec((_LI, _LJ), lambda i, j: (i, j)),
            pl.BlockSpec((_LI, _NHID), lambda i, j: (i, 0)),
            pl.BlockSpec((_LJ, _NHID), lambda i, j: (j, 0)),
        ],
        out_specs=pl.BlockSpec((1, 1), lambda i, j: (0, 0)),
        out_shape=jax.ShapeDtypeStruct((1, 1), jnp.float32),
    )(adj, gc_z, gc_z)

    return (x, loss[0, 0])


# speedup vs baseline: 15.0176x; 1.3216x over previous
import jax
import jax.numpy as jnp
from jax.experimental import pallas as pl

_N = 8192
_NFEAT = 256
_NHID = 64

_BM = 512
_LI = 512
_LJ = 8192


def _support_kernel(x_ref, w_ref, out_ref):
    out_ref[...] = jnp.dot(x_ref[...], w_ref[...],
                           preferred_element_type=jnp.float32)


def _gcz_kernel(adj_ref, sup_ref, b_ref, out_ref, half_ref):
    z = jnp.dot(adj_ref[...], sup_ref[...],
                preferred_element_type=jnp.float32) + b_ref[...]
    out_ref[...] = z
    half_ref[...] = 0.5 * z


def _loss_kernel(adj_ref, zi_ref, zj_ref, acc_ref):
    i = pl.program_id(0)
    j = pl.program_id(1)

    @pl.when((i == 0) & (j == 0))
    def _init():
        acc_ref[...] = jnp.zeros_like(acc_ref)

    zz = jax.lax.dot_general(
        zi_ref[...], zj_ref[...],
        dimension_numbers=(((1,), (1,)), ((), ())),
        preferred_element_type=jnp.float32)
    e = jnp.tanh(zz) + (1.0 - 2.0 * adj_ref[...])
    acc_ref[...] = acc_ref[...] + jnp.sum(e * e) * (0.25 / (_N * _N))


def kernel(x, adj, W, b):
    b2 = b.reshape(1, _NHID)

    support = pl.pallas_call(
        _support_kernel,
        out_shape=jax.ShapeDtypeStruct((_N, _NHID), jnp.float32),
    )(x, W)

    gc_z, gc_half = pl.pallas_call(
        _gcz_kernel,
        grid=(_N // _BM,),
        in_specs=[
            pl.BlockSpec((_BM, _N), lambda i: (i, 0)),
            pl.BlockSpec((_N, _NHID), lambda i: (0, 0)),
            pl.BlockSpec((1, _NHID), lambda i: (0, 0)),
        ],
        out_specs=[
            pl.BlockSpec((_BM, _NHID), lambda i: (i, 0)),
            pl.BlockSpec((_BM, _NHID), lambda i: (i, 0)),
        ],
        out_shape=[
            jax.ShapeDtypeStruct((_N, _NHID), jnp.float32),
            jax.ShapeDtypeStruct((_N, _NHID), jnp.float32),
        ],
    )(adj, support, b2)

    loss = pl.pallas_call(
        _loss_kernel,
        grid=(_N // _LI, _N // _LJ),
        in_specs=[
            pl.BlockSpec((_LI, _LJ), lambda i, j: (i, j)),
            pl.BlockSpec((_LI, _NHID), lambda i, j: (i, 0)),
            pl.BlockSpec((_LJ, _NHID), lambda i, j: (j, 0)),
        ],
        out_specs=pl.BlockSpec((1, 1), lambda i, j: (0, 0)),
        out_shape=jax.ShapeDtypeStruct((1, 1), jnp.float32),
    )(adj, gc_half, gc_z)

    return (x, loss[0, 0])
